# baseline (speedup 1.0000x reference)
import functools
import math

import jax
import jax.numpy as jnp
from jax import lax
from jax.experimental import pallas as pl
from jax.experimental.pallas import tpu as pltpu
from jax.experimental.pallas import tpu_sc as plsc

_LOG2E = math.log2(math.e)
_LN2 = math.log(2.0)


def _tc_body(t_ref, x_ref, out_ref, *, br, n):
    j = pl.program_id(0)
    x = x_ref[...]
    t = t_ref[...]
    a = lax.abs(x)
    e = jnp.exp2(a * (-_LOG2E))
    u = jnp.log2(1.0 + e)
    sp = jnp.maximum(x, 0.0) + _LN2 * u
    rowsum = jnp.sum(sp, axis=1, keepdims=True)
    good = t != 0
    psum = jnp.sum(jnp.where(good, rowsum, 0.0))
    pcnt = jnp.sum(jnp.where(good, 0.0, 1.0))

    @pl.when(j == 0)
    def _():
        out_ref[0, 0] = 0.0
        out_ref[0, 1] = 0.0

    out_ref[0, 0] += psum
    out_ref[0, 1] += pcnt


def _make_tc(m, n, br):
    return pl.pallas_call(
        functools.partial(_tc_body, br=br, n=n),
        grid=(m // br,),
        in_specs=[
            pl.BlockSpec((br, 1), lambda j: (j, 0)),
            pl.BlockSpec((br, n), lambda j: (j, 0)),
        ],
        out_specs=pl.BlockSpec(
            (1, 2), lambda j: (0, 0), memory_space=pltpu.SMEM
        ),
        out_shape=jax.ShapeDtypeStruct((1, 2), jnp.float32),
        compiler_params=pltpu.CompilerParams(
            dimension_semantics=("arbitrary",)
        ),
    )


def _make_sc_gather(m, n):
    mesh = plsc.VectorSubcoreMesh(core_axis_name="c", subcore_axis_name="s")
    nw = 32
    bpw = m // nw

    @functools.partial(
        pl.kernel,
        mesh=mesh,
        out_type=jax.ShapeDtypeStruct((nw, 16), jnp.float32),
        scratch_types=[
            pltpu.VMEM((bpw,), jnp.int32),
            pltpu.VMEM((bpw,), jnp.int32),
            pltpu.VMEM((bpw,), jnp.float32),
            pltpu.VMEM((16,), jnp.float32),
            pltpu.SemaphoreType.DMA,
        ],
    )
    def _g(x_hbm, t_hbm, out_hbm, t_v, idx_v, g_v, acc_v, sem):
        wid = lax.axis_index("s") * 2 + lax.axis_index("c")
        base = wid * bpw
        pltpu.sync_copy(t_hbm.at[pl.ds(base, bpw)], t_v)
        for k in range(bpw // 16):
            tv = t_v[pl.ds(k * 16, 16)]
            rows = (base + k * 16) + lax.broadcasted_iota(jnp.int32, (16,), 0)
            idx_v[pl.ds(k * 16, 16)] = rows * n + tv
        pltpu.async_copy(x_hbm.at[idx_v], g_v, sem).wait()
        acc = jnp.zeros((16,), jnp.float32)
        for k in range(bpw // 16):
            tv = t_v[pl.ds(k * 16, 16)]
            gv = g_v[pl.ds(k * 16, 16)]
            acc = acc + jnp.where(tv != 0, gv, 0.0)
        acc_v[...] = acc
        pltpu.sync_copy(acc_v, out_hbm.at[wid])

    return _g


def kernel(input, target):
    m, n = input.shape
    t32 = target.astype(jnp.int32)
    g_parts = _make_sc_gather(m, n)(input.reshape(m * n), t32)
    out = _make_tc(m, n, 32)(t32.reshape(m, 1), input)
    gsum = jnp.sum(g_parts)
    return (out[0, 0] - gsum) / jnp.maximum(out[0, 1], 1.0)


# speedup vs baseline: 1.8420x; 1.8420x over previous
import functools
import math

import jax
import jax.numpy as jnp
from jax.experimental import pallas as pl
from jax.experimental.pallas import tpu as pltpu

_LOG2E = math.log2(math.e)
_LN2 = math.log(2.0)


def _body(t_ref, x_ref, out_ref, *, br, n, nblocks):
    j = pl.program_id(0)

    x = x_ref[...]
    t = t_ref[...]
    a = jax.lax.abs(x)
    e = jnp.exp2(a * (-_LOG2E))
    u = jnp.log2(1.0 + e)
    sp = jnp.maximum(x, 0.0) + _LN2 * u
    col = jax.lax.broadcasted_iota(jnp.int32, (br, n), 1)
    contrib = sp - jnp.where(col == t, x, 0.0)
    rowsum = jnp.sum(contrib, axis=1, keepdims=True)
    good = t != 0
    psum = jnp.sum(jnp.where(good, rowsum, 0.0))
    pcnt = jnp.sum(jnp.where(good, 0.0, 1.0))

    @pl.when(j == 0)
    def _():
        out_ref[0, 0] = 0.0
        out_ref[0, 1] = 0.0

    out_ref[0, 0] += psum
    out_ref[0, 1] += pcnt

    @pl.when(j == nblocks - 1)
    def _():
        out_ref[0, 0] = out_ref[0, 0] / jnp.maximum(out_ref[0, 1], 1.0)


def kernel(input, target):
    m, n = input.shape
    br = 16
    nblocks = m // br
    t = target.astype(jnp.int32).reshape(m, 1)
    out = pl.pallas_call(
        functools.partial(_body, br=br, n=n, nblocks=nblocks),
        grid=(nblocks,),
        in_specs=[
            pl.BlockSpec((br, 1), lambda j: (j, 0)),
            pl.BlockSpec((br, n), lambda j: (j, 0)),
        ],
        out_specs=pl.BlockSpec(
            (1, 2), lambda j: (0, 0), memory_space=pltpu.SMEM
        ),
        out_shape=jax.ShapeDtypeStruct((1, 2), jnp.float32),
        compiler_params=pltpu.CompilerParams(
            dimension_semantics=("arbitrary",)
        ),
    )(t, input)
    return out[0, 0]


# speedup vs baseline: 1.8793x; 1.0203x over previous
import functools
import math

import jax
import jax.numpy as jnp
from jax.experimental import pallas as pl
from jax.experimental.pallas import tpu as pltpu

_LOG2E = math.log2(math.e)
_LN2 = math.log(2.0)


def _body(t_ref, x_ref, out_ref, *, br, n, nblocks):
    j = pl.program_id(0)

    x = x_ref[...]
    t = t_ref[...]
    a = jax.lax.abs(x)
    e = jnp.exp2(a * (-_LOG2E))
    u = jnp.log2(1.0 + e)
    sp = jnp.maximum(x, 0.0) + _LN2 * u
    col = jax.lax.broadcasted_iota(jnp.int32, (br, n), 1)
    contrib = sp - jnp.where(col == t, x, 0.0)
    rowsum = jnp.sum(contrib, axis=1, keepdims=True)
    good = t != 0
    psum = jnp.sum(jnp.where(good, rowsum, 0.0))
    pcnt = jnp.sum(jnp.where(good, 0.0, 1.0))

    @pl.when(j == 0)
    def _():
        out_ref[0, 0] = 0.0
        out_ref[0, 1] = 0.0

    out_ref[0, 0] += psum
    out_ref[0, 1] += pcnt

    @pl.when(j == nblocks - 1)
    def _():
        out_ref[0, 0] = out_ref[0, 0] / jnp.maximum(out_ref[0, 1], 1.0)


def kernel(input, target):
    m, n = input.shape
    br = 32
    nblocks = m // br
    t = target.astype(jnp.int32).reshape(m, 1)
    out = pl.pallas_call(
        functools.partial(_body, br=br, n=n, nblocks=nblocks),
        grid=(nblocks,),
        in_specs=[
            pl.BlockSpec((br, 1), lambda j: (j, 0)),
            pl.BlockSpec((br, n), lambda j: (j, 0)),
        ],
        out_specs=pl.BlockSpec(
            (1, 2), lambda j: (0, 0), memory_space=pltpu.SMEM
        ),
        out_shape=jax.ShapeDtypeStruct((1, 2), jnp.float32),
        compiler_params=pltpu.CompilerParams(
            dimension_semantics=("arbitrary",)
        ),
    )(t, input)
    return out[0, 0]
